# Initial kernel scaffold; baseline (speedup 1.0000x reference)
import jax
import jax.numpy as jnp
from jax.experimental import pallas as pl


def kernel(x, W1, W2, Wr):
    raise NotImplementedError("write your pallas kernel here")


# speedup vs baseline: 22.0117x; 22.0117x over previous
import jax
import jax.numpy as jnp
from jax.experimental import pallas as pl
from jax.experimental.pallas import tpu as pltpu

_B, _S, _H, _I, _E, _R, _K = 1, 2048, 2048, 8192, 4, 256, 2
_ALPHA = 16.0
_ER = _E * _R
_TI = 512


def _moe_body(x_ref, w1_ref, wr_ref, w2_ref, out_ref, aw_ref):
    i = pl.program_id(0)

    @pl.when(i == 0)
    def _prologue():
        xb = x_ref[...]
        h = jnp.dot(xb.astype(jnp.bfloat16), w1_ref[...],
                    preferred_element_type=jnp.float32)
        a = 0.5 * h * (1.0 + jax.lax.erf(h * 0.7071067811865476))
        logits = jnp.dot(xb, wr_ref[...], preferred_element_type=jnp.float32)
        m = jnp.max(logits, axis=1, keepdims=True)
        p = jnp.exp(logits - m)
        w = p / jnp.sum(p, axis=1, keepdims=True)
        col = jax.lax.broadcasted_iota(jnp.int32, (_S, _E), 1)
        rank = jnp.zeros((_S, _E), jnp.int32)
        for ep in range(_E):
            wep = w[:, ep:ep + 1]
            beats = (wep > w) | ((wep == w) & (ep < col))
            rank += beats.astype(jnp.int32)
        wsel = jnp.where(rank < _K, w, 0.0)
        c = wsel / (jnp.sum(wsel, axis=1, keepdims=True) + 1e-6) * (_ALPHA / _R)
        cfull = jnp.broadcast_to(c[:, :, None], (_S, _E, _R)).reshape(_S, _ER)
        aw_ref[...] = (a * cfull).astype(jnp.bfloat16)

    out_ref[...] = jnp.dot(aw_ref[...], w2_ref[...],
                           preferred_element_type=jnp.float32)


def kernel(x, W1, W2, Wr):
    xs = x.reshape(_S, _H)
    w1t = W1.reshape(_ER, _H).T.astype(jnp.bfloat16)
    w2t = jnp.transpose(W2, (0, 2, 1)).reshape(_ER, _I).astype(jnp.bfloat16)
    wrt = Wr.T
    out = pl.pallas_call(
        _moe_body,
        grid=(_I // _TI,),
        in_specs=[
            pl.BlockSpec((_S, _H), lambda i: (0, 0)),
            pl.BlockSpec((_H, _ER), lambda i: (0, 0)),
            pl.BlockSpec((_H, _E), lambda i: (0, 0)),
            pl.BlockSpec((_ER, _TI), lambda i: (0, i)),
        ],
        out_specs=pl.BlockSpec((_S, _TI), lambda i: (0, i)),
        out_shape=jax.ShapeDtypeStruct((_S, _I), jnp.float32),
        scratch_shapes=[pltpu.VMEM((_S, _ER), jnp.bfloat16)],
    )(xs, w1t, wrt, w2t)
    return out.reshape(_B, _S, _I)


# speedup vs baseline: 30.5339x; 1.3872x over previous
import jax
import jax.numpy as jnp
from jax.experimental import pallas as pl
from jax.experimental.pallas import tpu as pltpu

_B, _S, _H, _I, _E, _R, _K = 1, 2048, 2048, 8192, 4, 256, 2
_ALPHA = 16.0
_ER = _E * _R
_TI = 256

_DNT = (((1,), (1,)), ((), ()))


def _moe_body(x_ref, w1_ref, wr_ref, w2_ref, out_ref, aw_ref):
    i = pl.program_id(0)

    @pl.when(i == 0)
    def _prologue():
        xb = x_ref[...]
        h = jax.lax.dot_general(
            xb.astype(jnp.bfloat16), w1_ref[...].astype(jnp.bfloat16),
            _DNT, preferred_element_type=jnp.float32)
        a = 0.5 * h * (1.0 + jax.lax.erf(h * 0.7071067811865476))
        logits = jnp.dot(xb, wr_ref[...], preferred_element_type=jnp.float32)
        m = jnp.max(logits, axis=1, keepdims=True)
        p = jnp.exp(logits - m)
        w = p / jnp.sum(p, axis=1, keepdims=True)
        col = jax.lax.broadcasted_iota(jnp.int32, (_S, _E), 1)
        rank = jnp.zeros((_S, _E), jnp.int32)
        for ep in range(_E):
            wep = w[:, ep:ep + 1]
            beats = (wep > w) | ((wep == w) & (ep < col))
            rank += beats.astype(jnp.int32)
        wsel = jnp.where(rank < _K, w, 0.0)
        c = wsel / (jnp.sum(wsel, axis=1, keepdims=True) + 1e-6) * (_ALPHA / _R)
        cfull = jnp.broadcast_to(c[:, :, None], (_S, _E, _R)).reshape(_S, _ER)
        aw_ref[...] = (a * cfull).astype(jnp.bfloat16)

    w2b = w2_ref[...].astype(jnp.bfloat16)
    acc = jax.lax.dot_general(aw_ref[:, 0:_R], w2b[0], _DNT,
                              preferred_element_type=jnp.float32)
    for e in range(1, _E):
        acc += jax.lax.dot_general(aw_ref[:, e * _R:(e + 1) * _R], w2b[e],
                                   _DNT, preferred_element_type=jnp.float32)
    out_ref[...] = acc


def kernel(x, W1, W2, Wr):
    xs = x.reshape(_S, _H)
    w1r = W1.reshape(_ER, _H)
    wrt = Wr.T
    out = pl.pallas_call(
        _moe_body,
        grid=(_I // _TI,),
        in_specs=[
            pl.BlockSpec((_S, _H), lambda i: (0, 0)),
            pl.BlockSpec((_ER, _H), lambda i: (0, 0)),
            pl.BlockSpec((_H, _E), lambda i: (0, 0)),
            pl.BlockSpec((_E, _TI, _R), lambda i: (0, i, 0)),
        ],
        out_specs=pl.BlockSpec((_S, _TI), lambda i: (0, i)),
        out_shape=jax.ShapeDtypeStruct((_S, _I), jnp.float32),
        scratch_shapes=[pltpu.VMEM((_S, _ER), jnp.bfloat16)],
    )(xs, w1r, wrt, W2)
    return out.reshape(_B, _S, _I)


# speedup vs baseline: 32.0713x; 1.0504x over previous
import jax
import jax.numpy as jnp
from jax.experimental import pallas as pl
from jax.experimental.pallas import tpu as pltpu

_B, _S, _H, _I, _E, _R, _K = 1, 2048, 2048, 8192, 4, 256, 2
_ALPHA = 16.0
_ER = _E * _R
_TI = 512

_DNT = (((1,), (1,)), ((), ()))


def _moe_body(x_ref, w1_ref, wr_ref, w2_ref, out_ref, aw_ref):
    i = pl.program_id(0)

    @pl.when(i == 0)
    def _prologue():
        xb = x_ref[...]
        h = jax.lax.dot_general(
            xb.astype(jnp.bfloat16), w1_ref[...],
            _DNT, preferred_element_type=jnp.float32)
        a = 0.5 * h * (1.0 + jax.lax.erf(h * 0.7071067811865476))
        logits = jnp.dot(xb, wr_ref[...], preferred_element_type=jnp.float32)
        m = jnp.max(logits, axis=1, keepdims=True)
        p = jnp.exp(logits - m)
        w = p / jnp.sum(p, axis=1, keepdims=True)
        col = jax.lax.broadcasted_iota(jnp.int32, (_S, _E), 1)
        rank = jnp.zeros((_S, _E), jnp.int32)
        for ep in range(_E):
            wep = w[:, ep:ep + 1]
            beats = (wep > w) | ((wep == w) & (ep < col))
            rank += beats.astype(jnp.int32)
        wsel = jnp.where(rank < _K, w, 0.0)
        c = wsel / (jnp.sum(wsel, axis=1, keepdims=True) + 1e-6) * (_ALPHA / _R)
        cfull = jnp.broadcast_to(c[:, :, None], (_S, _E, _R)).reshape(_S, _ER)
        aw_ref[...] = (a * cfull).astype(jnp.bfloat16)

    w2b = w2_ref[...].astype(jnp.bfloat16)
    acc = jax.lax.dot_general(aw_ref[:, 0:_R], w2b[0], _DNT,
                              preferred_element_type=jnp.float32)
    for e in range(1, _E):
        acc += jax.lax.dot_general(aw_ref[:, e * _R:(e + 1) * _R], w2b[e],
                                   _DNT, preferred_element_type=jnp.float32)
    out_ref[...] = acc


def kernel(x, W1, W2, Wr):
    xs = x.reshape(_S, _H)
    w1r = W1.reshape(_ER, _H).astype(jnp.bfloat16)
    wrt = Wr.T
    out = pl.pallas_call(
        _moe_body,
        grid=(_I // _TI,),
        in_specs=[
            pl.BlockSpec((_S, _H), lambda i: (0, 0)),
            pl.BlockSpec((_ER, _H), lambda i: (0, 0)),
            pl.BlockSpec((_H, _E), lambda i: (0, 0)),
            pl.BlockSpec((_E, _TI, _R), lambda i: (0, i, 0)),
        ],
        out_specs=pl.BlockSpec((_S, _TI), lambda i: (0, i)),
        out_shape=jax.ShapeDtypeStruct((_S, _I), jnp.float32),
        scratch_shapes=[pltpu.VMEM((_S, _ER), jnp.bfloat16)],
    )(xs, w1r, wrt, W2)
    return out.reshape(_B, _S, _I)


# speedup vs baseline: 36.5623x; 1.1400x over previous
import jax
import jax.numpy as jnp
from jax.experimental import pallas as pl
from jax.experimental.pallas import tpu as pltpu

_B, _S, _H, _I, _E, _R, _K = 1, 2048, 2048, 8192, 4, 256, 2
_ALPHA = 16.0
_ER = _E * _R
_TI = 512

_DNT = (((1,), (1,)), ((), ()))


def _moe_body(x_ref, w1_ref, wr_ref, w2_ref, out_ref, aw_ref):
    i = pl.program_id(0)

    @pl.when(i == 0)
    def _prologue():
        xb = x_ref[...]
        logits = jnp.dot(xb, wr_ref[...], preferred_element_type=jnp.float32)
        m = jnp.max(logits, axis=1, keepdims=True)
        p = jnp.exp(logits - m)
        w = p / jnp.sum(p, axis=1, keepdims=True)
        col = jax.lax.broadcasted_iota(jnp.int32, (_S, _E), 1)
        rank = jnp.zeros((_S, _E), jnp.int32)
        for ep in range(_E):
            wep = w[:, ep:ep + 1]
            beats = (wep > w) | ((wep == w) & (ep < col))
            rank += beats.astype(jnp.int32)
        wsel = jnp.where(rank < _K, w, 0.0)
        c = wsel / (jnp.sum(wsel, axis=1, keepdims=True) + 1e-6) * (_ALPHA / _R)
        xb16 = xb.astype(jnp.bfloat16)
        for e in range(_E):
            sl = slice(e * _R, (e + 1) * _R)
            h = jax.lax.dot_general(xb16, w1_ref[sl, :], _DNT,
                                    preferred_element_type=jnp.float32)
            a = 0.5 * h * (1.0 + jax.lax.erf(h * 0.7071067811865476))
            aw_ref[:, sl] = (a * c[:, e:e + 1]).astype(jnp.bfloat16)

    acc = jax.lax.dot_general(
        aw_ref[:, 0:_R], w2_ref[0].astype(jnp.bfloat16), _DNT,
        preferred_element_type=jnp.float32)
    for e in range(1, _E):
        acc += jax.lax.dot_general(
            aw_ref[:, e * _R:(e + 1) * _R], w2_ref[e].astype(jnp.bfloat16),
            _DNT, preferred_element_type=jnp.float32)
    out_ref[...] = acc


def kernel(x, W1, W2, Wr):
    xs = x.reshape(_S, _H)
    w1r = W1.reshape(_ER, _H).astype(jnp.bfloat16)
    wrt = Wr.T
    out = pl.pallas_call(
        _moe_body,
        grid=(_I // _TI,),
        in_specs=[
            pl.BlockSpec((_S, _H), lambda i: (0, 0)),
            pl.BlockSpec((_ER, _H), lambda i: (0, 0)),
            pl.BlockSpec((_H, _E), lambda i: (0, 0)),
            pl.BlockSpec((_E, _TI, _R), lambda i: (0, i, 0)),
        ],
        out_specs=pl.BlockSpec((_S, _TI), lambda i: (0, i)),
        out_shape=jax.ShapeDtypeStruct((_S, _I), jnp.float32),
        scratch_shapes=[pltpu.VMEM((_S, _ER), jnp.bfloat16)],
    )(xs, w1r, wrt, W2)
    return out.reshape(_B, _S, _I)
